# baseline (speedup 1.0000x reference)
import functools

import jax
import jax.numpy as jnp
from jax import lax
from jax.experimental import pallas as pl
from jax.experimental.pallas import tpu as pltpu
from jax.experimental.pallas import tpu_sc as plsc

NUM_FEATURES = 26
B = 16384
D = 16

_info = plsc.get_sparse_core_info()
NC, NS = _info.num_cores, _info.num_subcores
NW = NC * NS
BPW = B // NW


def _emb_body(*refs):
    feats = refs[:NUM_FEATURES]
    tables = refs[NUM_FEATURES:2 * NUM_FEATURES]
    out = refs[2 * NUM_FEATURES]
    idx_v, rows_v, sem = refs[2 * NUM_FEATURES + 1:]

    wid = lax.axis_index("s") * NC + lax.axis_index("c")
    base = wid * BPW

    for f in range(NUM_FEATURES):
        pltpu.sync_copy(feats[f].at[pl.ds(base, BPW)], idx_v)
        pltpu.async_copy(tables[f].at[idx_v], rows_v, sem).wait()
        pltpu.sync_copy(rows_v, out.at[pl.ds(base, BPW), f])


_emb_call = functools.partial(
    pl.kernel,
    out_type=jax.ShapeDtypeStruct((B, NUM_FEATURES, D), jnp.float32),
    mesh=plsc.VectorSubcoreMesh(core_axis_name="c", subcore_axis_name="s"),
    scratch_types=[
        pltpu.VMEM((BPW,), jnp.int32),
        pltpu.VMEM((BPW, D), jnp.float32),
        pltpu.SemaphoreType.DMA,
    ],
    compiler_params=pltpu.CompilerParams(use_tc_tiling_on_sc=False),
)(_emb_body)


def kernel(f0, f1, f2, f3, f4, f5, f6, f7, f8, f9, f10, f11, f12, f13, f14,
           f15, f16, f17, f18, f19, f20, f21, f22, f23, f24, f25,
           W_f0, W_f1, W_f2, W_f3, W_f4, W_f5, W_f6, W_f7, W_f8, W_f9, W_f10,
           W_f11, W_f12, W_f13, W_f14, W_f15, W_f16, W_f17, W_f18, W_f19,
           W_f20, W_f21, W_f22, W_f23, W_f24, W_f25):
    feats = [f0, f1, f2, f3, f4, f5, f6, f7, f8, f9, f10, f11, f12, f13, f14,
             f15, f16, f17, f18, f19, f20, f21, f22, f23, f24, f25]
    tables = [W_f0, W_f1, W_f2, W_f3, W_f4, W_f5, W_f6, W_f7, W_f8, W_f9,
              W_f10, W_f11, W_f12, W_f13, W_f14, W_f15, W_f16, W_f17, W_f18,
              W_f19, W_f20, W_f21, W_f22, W_f23, W_f24, W_f25]
    return _emb_call(*feats, *tables)


# speedup vs baseline: 1.0284x; 1.0284x over previous
import functools

import jax
import jax.numpy as jnp
from jax import lax
from jax.experimental import pallas as pl
from jax.experimental.pallas import tpu as pltpu
from jax.experimental.pallas import tpu_sc as plsc

NUM_FEATURES = 26
B = 16384
D = 16

_info = plsc.get_sparse_core_info()
NC, NS = _info.num_cores, _info.num_subcores
NW = NC * NS
BPW = B // NW


NBUF = 4


def _emb_body(*refs):
    feats = refs[:NUM_FEATURES]
    tables = refs[NUM_FEATURES:2 * NUM_FEATURES]
    out = refs[2 * NUM_FEATURES]
    rest = refs[2 * NUM_FEATURES + 1:]
    idx_all = rest[0]
    rows = rest[1:1 + NBUF]
    isem = rest[1 + NBUF]
    gsems = rest[2 + NBUF:2 + 2 * NBUF]
    wsems = rest[2 + 2 * NBUF:2 + 3 * NBUF]

    wid = lax.axis_index("s") * NC + lax.axis_index("c")
    base = wid * BPW

    idx_copies = [
        pltpu.make_async_copy(feats[f].at[pl.ds(base, BPW)], idx_all.at[f], isem)
        for f in range(NUM_FEATURES)
    ]
    for c in idx_copies:
        c.start()
    for c in idx_copies:
        c.wait()

    def gather(f):
        b = f % NBUF
        pltpu.make_async_copy(tables[f].at[idx_all.at[f]], rows[b],
                              gsems[b]).start()

    def write(f):
        b = f % NBUF
        return pltpu.make_async_copy(rows[b], out.at[pl.ds(base, BPW), f],
                                     wsems[b])

    for f in range(NBUF):
        gather(f)

    pending_writes = {}
    for f in range(NUM_FEATURES):
        b = f % NBUF
        pltpu.make_async_copy(tables[f].at[idx_all.at[f]], rows[b],
                              gsems[b]).wait()
        w = write(f)
        w.start()
        pending_writes[b] = w
        nf = f + NBUF
        if nf < NUM_FEATURES:
            pending_writes[b].wait()
            del pending_writes[b]
            gather(nf)
    for w in pending_writes.values():
        w.wait()


_emb_call = functools.partial(
    pl.kernel,
    out_type=jax.ShapeDtypeStruct((B, NUM_FEATURES, D), jnp.float32),
    mesh=plsc.VectorSubcoreMesh(core_axis_name="c", subcore_axis_name="s"),
    scratch_types=(
        [pltpu.VMEM((NUM_FEATURES, BPW), jnp.int32)]
        + [pltpu.VMEM((BPW, D), jnp.float32) for _ in range(NBUF)]
        + [pltpu.SemaphoreType.DMA] * (1 + 2 * NBUF)
    ),
    compiler_params=pltpu.CompilerParams(use_tc_tiling_on_sc=False),
)(_emb_body)


def kernel(f0, f1, f2, f3, f4, f5, f6, f7, f8, f9, f10, f11, f12, f13, f14,
           f15, f16, f17, f18, f19, f20, f21, f22, f23, f24, f25,
           W_f0, W_f1, W_f2, W_f3, W_f4, W_f5, W_f6, W_f7, W_f8, W_f9, W_f10,
           W_f11, W_f12, W_f13, W_f14, W_f15, W_f16, W_f17, W_f18, W_f19,
           W_f20, W_f21, W_f22, W_f23, W_f24, W_f25):
    feats = [f0, f1, f2, f3, f4, f5, f6, f7, f8, f9, f10, f11, f12, f13, f14,
             f15, f16, f17, f18, f19, f20, f21, f22, f23, f24, f25]
    tables = [W_f0, W_f1, W_f2, W_f3, W_f4, W_f5, W_f6, W_f7, W_f8, W_f9,
              W_f10, W_f11, W_f12, W_f13, W_f14, W_f15, W_f16, W_f17, W_f18,
              W_f19, W_f20, W_f21, W_f22, W_f23, W_f24, W_f25]
    return _emb_call(*feats, *tables)


# speedup vs baseline: 4.0169x; 3.9061x over previous
import functools

import jax
import jax.numpy as jnp
from jax import lax
from jax.experimental import pallas as pl
from jax.experimental.pallas import tpu as pltpu
from jax.experimental.pallas import tpu_sc as plsc

NUM_FEATURES = 26
B = 16384
D = 16
V = 100001

_info = plsc.get_sparse_core_info()
NC, NS = _info.num_cores, _info.num_subcores
BH = B // NC
CH = 512
NCH = BH // CH


def _emb_body(*refs):
    feats = refs[:NUM_FEATURES]
    tables = refs[NUM_FEATURES:2 * NUM_FEATURES]
    out = refs[2 * NUM_FEATURES]
    vrow, vidx, vout, rsem, isem, osem = refs[2 * NUM_FEATURES + 1:]

    cid = lax.axis_index("c")
    sid = lax.axis_index("s")
    zeros16 = jnp.zeros((16,), jnp.int32)
    hbase = cid * BH

    for f in range(NUM_FEATURES):
        pltpu.make_async_copy(
            tables[f].at[pl.ds(sid, 1), :], vrow, rsem).start()
        pltpu.make_async_copy(
            feats[f].at[pl.ds(hbase, BH)], vidx, isem).start()
        pltpu.make_async_copy(
            tables[f].at[pl.ds(sid, 1), :], vrow, rsem).wait()
        pltpu.make_async_copy(
            feats[f].at[pl.ds(hbase, BH)], vidx, isem).wait()

        def chunk_body(ch, carry):
            base = ch * CH
            for i in range(CH // 16):
                g = plsc.load_gather(
                    vrow, [zeros16, vidx[pl.ds(base + i * 16, 16)]])
                vout[0, 0, pl.ds(i * 16, 16)] = g
            pltpu.make_async_copy(
                vout,
                out.at[pl.ds(f, 1), pl.ds(sid, 1), pl.ds(hbase + base, CH)],
                osem).start()
            pltpu.make_async_copy(
                vout,
                out.at[pl.ds(0, 1), pl.ds(0, 1), pl.ds(0, CH)],
                osem).wait()
            return carry

        lax.fori_loop(0, NCH, chunk_body, 0, unroll=False)


_emb_call = functools.partial(
    pl.kernel,
    out_type=jax.ShapeDtypeStruct((NUM_FEATURES, D, B), jnp.float32),
    mesh=plsc.VectorSubcoreMesh(core_axis_name="c", subcore_axis_name="s"),
    scratch_types=[
        pltpu.VMEM((1, V), jnp.float32),
        pltpu.VMEM((BH,), jnp.int32),
        pltpu.VMEM((1, 1, CH), jnp.float32),
        pltpu.SemaphoreType.DMA,
        pltpu.SemaphoreType.DMA,
        pltpu.SemaphoreType.DMA,
    ],
    compiler_params=pltpu.CompilerParams(
        use_tc_tiling_on_sc=True, needs_layout_passes=False),
)(_emb_body)


def kernel(f0, f1, f2, f3, f4, f5, f6, f7, f8, f9, f10, f11, f12, f13, f14,
           f15, f16, f17, f18, f19, f20, f21, f22, f23, f24, f25,
           W_f0, W_f1, W_f2, W_f3, W_f4, W_f5, W_f6, W_f7, W_f8, W_f9, W_f10,
           W_f11, W_f12, W_f13, W_f14, W_f15, W_f16, W_f17, W_f18, W_f19,
           W_f20, W_f21, W_f22, W_f23, W_f24, W_f25):
    feats = [f0, f1, f2, f3, f4, f5, f6, f7, f8, f9, f10, f11, f12, f13, f14,
             f15, f16, f17, f18, f19, f20, f21, f22, f23, f24, f25]
    tables = [W_f0, W_f1, W_f2, W_f3, W_f4, W_f5, W_f6, W_f7, W_f8, W_f9,
              W_f10, W_f11, W_f12, W_f13, W_f14, W_f15, W_f16, W_f17, W_f18,
              W_f19, W_f20, W_f21, W_f22, W_f23, W_f24, W_f25]
    out_t = _emb_call(*feats, *[w.T for w in tables])
    return out_t.transpose(2, 0, 1)


# speedup vs baseline: 5.5074x; 1.3711x over previous
import functools

import jax
import jax.numpy as jnp
from jax import lax
from jax.experimental import pallas as pl
from jax.experimental.pallas import tpu as pltpu
from jax.experimental.pallas import tpu_sc as plsc

NUM_FEATURES = 26
B = 16384
D = 16
V = 100001

_info = plsc.get_sparse_core_info()
NC, NS = _info.num_cores, _info.num_subcores
BH = B // NC
UNROLL = 16
NIT = BH // (16 * UNROLL)


def _emb_body(*refs):
    feats = refs[:NUM_FEATURES]
    tables = refs[NUM_FEATURES:2 * NUM_FEATURES]
    out = refs[2 * NUM_FEATURES]
    vrow, vidx, vout, rsem, isem, osem0, osem1 = refs[2 * NUM_FEATURES + 1:]

    cid = lax.axis_index("c")
    sid = lax.axis_index("s")
    zeros16 = jnp.zeros((16,), jnp.int32)
    hbase = cid * BH

    def stage(f):
        pltpu.make_async_copy(
            tables[f].at[pl.ds(sid, 1), :], vrow, rsem).start()
        pltpu.make_async_copy(
            feats[f].at[pl.ds(hbase, BH)], vidx, isem).start()

    def write(f, p):
        return pltpu.make_async_copy(
            vout.at[pl.ds(p, 1)],
            out.at[pl.ds(f, 1), pl.ds(sid, 1), pl.ds(hbase, BH)],
            osem0 if p == 0 else osem1)

    stage(0)
    for f in range(NUM_FEATURES):
        p = f % 2
        pltpu.make_async_copy(
            tables[0].at[pl.ds(0, 1), :], vrow, rsem).wait()
        pltpu.make_async_copy(
            feats[0].at[pl.ds(0, BH)], vidx, isem).wait()
        if f >= 2:
            write(0, p).wait()

        def chunk_body(it, carry):
            base = it * (16 * UNROLL)
            for i in range(UNROLL):
                g = plsc.load_gather(
                    vrow, [zeros16, vidx[pl.ds(base + i * 16, 16)]])
                vout[p, 0, pl.ds(base + i * 16, 16)] = g
            return carry

        lax.fori_loop(0, NIT, chunk_body, 0, unroll=False)

        if f + 1 < NUM_FEATURES:
            stage(f + 1)
        write(f, p).start()

    write(0, 0).wait()
    write(0, 1).wait()


_emb_call = functools.partial(
    pl.kernel,
    out_type=jax.ShapeDtypeStruct((NUM_FEATURES, D, B), jnp.float32),
    mesh=plsc.VectorSubcoreMesh(core_axis_name="c", subcore_axis_name="s"),
    scratch_types=[
        pltpu.VMEM((1, V), jnp.float32),
        pltpu.VMEM((BH,), jnp.int32),
        pltpu.VMEM((2, 1, BH), jnp.float32),
        pltpu.SemaphoreType.DMA,
        pltpu.SemaphoreType.DMA,
        pltpu.SemaphoreType.DMA,
        pltpu.SemaphoreType.DMA,
    ],
    compiler_params=pltpu.CompilerParams(
        use_tc_tiling_on_sc=True, needs_layout_passes=False),
)(_emb_body)


def kernel(f0, f1, f2, f3, f4, f5, f6, f7, f8, f9, f10, f11, f12, f13, f14,
           f15, f16, f17, f18, f19, f20, f21, f22, f23, f24, f25,
           W_f0, W_f1, W_f2, W_f3, W_f4, W_f5, W_f6, W_f7, W_f8, W_f9, W_f10,
           W_f11, W_f12, W_f13, W_f14, W_f15, W_f16, W_f17, W_f18, W_f19,
           W_f20, W_f21, W_f22, W_f23, W_f24, W_f25):
    feats = [f0, f1, f2, f3, f4, f5, f6, f7, f8, f9, f10, f11, f12, f13, f14,
             f15, f16, f17, f18, f19, f20, f21, f22, f23, f24, f25]
    tables = [W_f0, W_f1, W_f2, W_f3, W_f4, W_f5, W_f6, W_f7, W_f8, W_f9,
              W_f10, W_f11, W_f12, W_f13, W_f14, W_f15, W_f16, W_f17, W_f18,
              W_f19, W_f20, W_f21, W_f22, W_f23, W_f24, W_f25]
    out_t = _emb_call(*feats, *[w.T for w in tables])
    return out_t.transpose(2, 0, 1)


# speedup vs baseline: 7.4237x; 1.3479x over previous
import functools

import jax
import jax.numpy as jnp
from jax import lax
from jax.experimental import pallas as pl
from jax.experimental.pallas import tpu as pltpu
from jax.experimental.pallas import tpu_sc as plsc

NUM_FEATURES = 26
B = 16384
D = 16
V = 100001

_info = plsc.get_sparse_core_info()
NC, NS = _info.num_cores, _info.num_subcores
NSTEP = NUM_FEATURES // NC
NBLK = 4
BLK = B // NBLK
UNROLL = 16
NIT = BLK // (16 * UNROLL)


def _emb_body(*refs):
    feats = refs[:NUM_FEATURES]
    tables = refs[NUM_FEATURES:2 * NUM_FEATURES]
    out = refs[2 * NUM_FEATURES]
    (vrow, vidx, vout, vpre, rsem, isem, psem,
     osem0, osem1) = refs[2 * NUM_FEATURES + 1:]

    cid = lax.axis_index("c")
    sid = lax.axis_index("s")
    zeros16 = jnp.zeros((16,), jnp.int32)

    for t in range(NUM_FEATURES):
        pltpu.make_async_copy(
            tables[t].at[pl.ds(0, 1), pl.ds(0, 128)], vpre, psem).start()
        pltpu.make_async_copy(
            feats[t].at[pl.ds(0, 128)], vidx.at[pl.ds(0, 128)], psem).start()
    for t in range(NUM_FEATURES):
        pltpu.make_async_copy(
            tables[t].at[pl.ds(0, 1), pl.ds(0, 128)], vpre, psem).wait()
        pltpu.make_async_copy(
            feats[t].at[pl.ds(0, 128)], vidx.at[pl.ds(0, 128)], psem).wait()

    def stage(j):
        for cc in range(NC):
            @pl.when(cid == cc)
            def _(cc=cc):
                t = j * NC + cc
                pltpu.make_async_copy(
                    tables[t].at[pl.ds(sid, 1), :], vrow, rsem).start()
                pltpu.make_async_copy(feats[t], vidx, isem).start()

    def write(j, b):
        return pltpu.make_async_copy(
            vout.at[pl.ds(b % 2, 1)],
            out.at[pl.ds(j * NC + cid, 1), pl.ds(sid, 1),
                   pl.ds(b * BLK, BLK)],
            osem0 if b % 2 == 0 else osem1)

    stage(0)
    flat = 0
    pending = []
    for j in range(NSTEP):
        pltpu.make_async_copy(
            tables[0].at[pl.ds(0, 1), :], vrow, rsem).wait()
        pltpu.make_async_copy(feats[0], vidx, isem).wait()

        for b in range(NBLK):
            if flat >= 2:
                pending.pop(0).wait()

            def chunk_body(it, carry, b=b):
                base = it * (16 * UNROLL)
                for i in range(UNROLL):
                    g = plsc.load_gather(
                        vrow,
                        [zeros16,
                         vidx[pl.ds(b * BLK + base + i * 16, 16)]])
                    vout[b % 2, 0, pl.ds(base + i * 16, 16)] = g
                return carry

            lax.fori_loop(0, NIT, chunk_body, 0, unroll=False)

            if b == NBLK - 1 and j + 1 < NSTEP:
                stage(j + 1)
            w = write(j, b)
            w.start()
            pending.append(w)
            flat += 1

    for w in pending:
        w.wait()


_emb_call = functools.partial(
    pl.kernel,
    out_type=jax.ShapeDtypeStruct((NUM_FEATURES, D, B), jnp.float32),
    mesh=plsc.VectorSubcoreMesh(core_axis_name="c", subcore_axis_name="s"),
    scratch_types=[
        pltpu.VMEM((1, V), jnp.float32),
        pltpu.VMEM((B,), jnp.int32),
        pltpu.VMEM((2, 1, BLK), jnp.float32),
        pltpu.VMEM((1, 128), jnp.float32),
        pltpu.SemaphoreType.DMA,
        pltpu.SemaphoreType.DMA,
        pltpu.SemaphoreType.DMA,
        pltpu.SemaphoreType.DMA,
        pltpu.SemaphoreType.DMA,
    ],
    compiler_params=pltpu.CompilerParams(
        use_tc_tiling_on_sc=True, needs_layout_passes=False),
)(_emb_body)


def kernel(f0, f1, f2, f3, f4, f5, f6, f7, f8, f9, f10, f11, f12, f13, f14,
           f15, f16, f17, f18, f19, f20, f21, f22, f23, f24, f25,
           W_f0, W_f1, W_f2, W_f3, W_f4, W_f5, W_f6, W_f7, W_f8, W_f9, W_f10,
           W_f11, W_f12, W_f13, W_f14, W_f15, W_f16, W_f17, W_f18, W_f19,
           W_f20, W_f21, W_f22, W_f23, W_f24, W_f25):
    feats = [f0, f1, f2, f3, f4, f5, f6, f7, f8, f9, f10, f11, f12, f13, f14,
             f15, f16, f17, f18, f19, f20, f21, f22, f23, f24, f25]
    tables = [W_f0, W_f1, W_f2, W_f3, W_f4, W_f5, W_f6, W_f7, W_f8, W_f9,
              W_f10, W_f11, W_f12, W_f13, W_f14, W_f15, W_f16, W_f17, W_f18,
              W_f19, W_f20, W_f21, W_f22, W_f23, W_f24, W_f25]
    out_t = _emb_call(*feats, *[w.T for w in tables])
    return out_t.transpose(2, 0, 1)


# speedup vs baseline: 7.4308x; 1.0010x over previous
import functools

import jax
import jax.numpy as jnp
from jax import lax
from jax.experimental import pallas as pl
from jax.experimental.pallas import tpu as pltpu
from jax.experimental.pallas import tpu_sc as plsc

NUM_FEATURES = 26
B = 16384
D = 16
V = 100001

_info = plsc.get_sparse_core_info()
NC, NS = _info.num_cores, _info.num_subcores
NSTEP = NUM_FEATURES // NC
NBLK = 4
BLK = B // NBLK
UNROLL = 16
NIT = BLK // (16 * UNROLL)


def _emb_body(*refs):
    feats = refs[:NUM_FEATURES]
    tables = refs[NUM_FEATURES:2 * NUM_FEATURES]
    out = refs[2 * NUM_FEATURES]
    (vrow, vidx, vout, vpre, rsem, isem, psem,
     osem0, osem1, osem2) = refs[2 * NUM_FEATURES + 1:]

    cid = lax.axis_index("c")
    sid = lax.axis_index("s")
    zeros16 = jnp.zeros((16,), jnp.int32)

    for t in range(NUM_FEATURES):
        pltpu.make_async_copy(
            tables[t].at[pl.ds(0, 1), pl.ds(0, 128)], vpre, psem).start()
        pltpu.make_async_copy(
            feats[t].at[pl.ds(0, 128)], vidx.at[pl.ds(0, 128)], psem).start()
    for t in range(NUM_FEATURES):
        pltpu.make_async_copy(
            tables[t].at[pl.ds(0, 1), pl.ds(0, 128)], vpre, psem).wait()
        pltpu.make_async_copy(
            feats[t].at[pl.ds(0, 128)], vidx.at[pl.ds(0, 128)], psem).wait()

    def stage(j):
        for cc in range(NC):
            @pl.when(cid == cc)
            def _(cc=cc):
                t = j * NC + cc
                pltpu.make_async_copy(
                    tables[t].at[pl.ds(sid, 1), :], vrow, rsem).start()
                pltpu.make_async_copy(feats[t], vidx, isem).start()

    osems = (osem0, osem1, osem2)

    def write(j, b, slot):
        return pltpu.make_async_copy(
            vout.at[pl.ds(slot, 1)],
            out.at[pl.ds(j * NC + cid, 1), pl.ds(sid, 1),
                   pl.ds(b * BLK, BLK)],
            osems[slot])

    stage(0)
    flat = 0
    pending = []
    for j in range(NSTEP):
        pltpu.make_async_copy(
            tables[0].at[pl.ds(0, 1), :], vrow, rsem).wait()
        pltpu.make_async_copy(feats[0], vidx, isem).wait()

        for b in range(NBLK):
            slot = flat % 3
            if flat >= 3:
                pending.pop(0).wait()

            def chunk_body(it, carry, b=b, slot=slot):
                base = it * (16 * UNROLL)
                for i in range(UNROLL):
                    g = plsc.load_gather(
                        vrow,
                        [zeros16,
                         vidx[pl.ds(b * BLK + base + i * 16, 16)]])
                    vout[slot, 0, pl.ds(base + i * 16, 16)] = g
                return carry

            lax.fori_loop(0, NIT, chunk_body, 0, unroll=False)

            if b == NBLK - 1 and j + 1 < NSTEP:
                stage(j + 1)
            w = write(j, b, slot)
            w.start()
            pending.append(w)
            flat += 1

    for w in pending:
        w.wait()


_emb_call = functools.partial(
    pl.kernel,
    out_type=jax.ShapeDtypeStruct((NUM_FEATURES, D, B), jnp.float32),
    mesh=plsc.VectorSubcoreMesh(core_axis_name="c", subcore_axis_name="s"),
    scratch_types=[
        pltpu.VMEM((1, V), jnp.float32),
        pltpu.VMEM((B,), jnp.int32),
        pltpu.VMEM((3, 1, BLK), jnp.float32),
        pltpu.VMEM((1, 128), jnp.float32),
        pltpu.SemaphoreType.DMA,
        pltpu.SemaphoreType.DMA,
        pltpu.SemaphoreType.DMA,
        pltpu.SemaphoreType.DMA,
        pltpu.SemaphoreType.DMA,
        pltpu.SemaphoreType.DMA,
    ],
    compiler_params=pltpu.CompilerParams(
        use_tc_tiling_on_sc=True, needs_layout_passes=False),
)(_emb_body)


def kernel(f0, f1, f2, f3, f4, f5, f6, f7, f8, f9, f10, f11, f12, f13, f14,
           f15, f16, f17, f18, f19, f20, f21, f22, f23, f24, f25,
           W_f0, W_f1, W_f2, W_f3, W_f4, W_f5, W_f6, W_f7, W_f8, W_f9, W_f10,
           W_f11, W_f12, W_f13, W_f14, W_f15, W_f16, W_f17, W_f18, W_f19,
           W_f20, W_f21, W_f22, W_f23, W_f24, W_f25):
    feats = [f0, f1, f2, f3, f4, f5, f6, f7, f8, f9, f10, f11, f12, f13, f14,
             f15, f16, f17, f18, f19, f20, f21, f22, f23, f24, f25]
    tables = [W_f0, W_f1, W_f2, W_f3, W_f4, W_f5, W_f6, W_f7, W_f8, W_f9,
              W_f10, W_f11, W_f12, W_f13, W_f14, W_f15, W_f16, W_f17, W_f18,
              W_f19, W_f20, W_f21, W_f22, W_f23, W_f24, W_f25]
    out_t = _emb_call(*feats, *[w.T for w in tables])
    return out_t.transpose(2, 0, 1)
